# baseline (speedup 1.0000x reference)
import functools

import jax
import jax.numpy as jnp
import numpy as np
from jax.experimental import pallas as pl
from jax.experimental.pallas import tpu as pltpu
from jax.experimental.pallas import tpu_sc as plsc

N_NODES = 10000
N_EDGES = 320000
N_HALF = N_EDGES // 2
HIDDEN = 128
HWORDS = HIDDEN // 2
EDGE_DIM = 16
OUT_DIM = 2 * HIDDEN
BN_EPS = 1e-5

NUM_CORES = 2
NUM_SUBCORES = 16
NUM_WORKERS = NUM_CORES * NUM_SUBCORES

GATHER_W = 128
EDGE_TILE = 1280
HTILE = EDGE_TILE // 2
N_TILES = N_EDGES // EDGE_TILE

CHUNK = 128
HALF_CHUNKS = N_HALF // CHUNK
CHUNKS_PER_WORKER = HALF_CHUNKS // NUM_WORKERS
REM_CHUNKS = HALF_CHUNKS - CHUNKS_PER_WORKER * NUM_WORKERS
N_NODES_PAD = 10240
ROWS_PER_SUBCORE = N_NODES_PAD // NUM_SUBCORES
ZROWS = 128

_PERM = np.concatenate([np.arange(0, 128, 2), np.arange(1, 128, 2),
                        np.arange(128, 256, 2), np.arange(129, 256, 2)])


def _sc_mesh():
    return plsc.VectorSubcoreMesh(core_axis_name="core",
                                  subcore_axis_name="subcore")


def _sc_gather(packed, src_idx, dst_idx):
    out_t = jax.ShapeDtypeStruct((N_EDGES, HWORDS), jnp.int32)

    @functools.partial(
        pl.kernel, out_type=(out_t, out_t), mesh=_sc_mesh(),
        compiler_params=pltpu.CompilerParams(use_tc_tiling_on_sc=False),
    )
    def k(nf_hbm, si_hbm, di_hbm, os_hbm, od_hbm):
        def body(si_v, di_v, os_v, od_v):
            pltpu.sync_copy(nf_hbm.at[si_v.at[0]], os_v)
            pltpu.sync_copy(nf_hbm.at[di_v.at[0]], od_v)

        pltpu.emit_pipeline(
            body,
            grid=(N_EDGES // GATHER_W,),
            in_specs=[
                pl.BlockSpec((1, GATHER_W), lambda i: (0, i)),
                pl.BlockSpec((1, GATHER_W), lambda i: (0, i)),
            ],
            out_specs=[
                pl.BlockSpec((GATHER_W, HWORDS), lambda i: (i, 0)),
                pl.BlockSpec((GATHER_W, HWORDS), lambda i: (i, 0)),
            ],
            core_axis_name=("core", "subcore"),
            dimension_semantics=(pltpu.PARALLEL,),
        )(si_hbm, di_hbm, os_hbm, od_hbm)

    src_raw, dst_raw = k(packed, src_idx, dst_idx)
    return (src_raw.reshape(N_HALF, 2 * HWORDS),
            dst_raw.reshape(N_HALF, 2 * HWORDS))


def _unpack_bf16(xi):
    ev = pltpu.bitcast(xi << 16, jnp.float32).astype(jnp.bfloat16)
    od = pltpu.bitcast(xi & jnp.int32(-65536), jnp.float32).astype(
        jnp.bfloat16)
    return ev, od


def _z_pair(src_ref, dst_ref, ef0_ref, ef1_ref, w12_ref, w3_ref, b_ref):
    sev, sod = _unpack_bf16(src_ref[...])
    dev, dod = _unpack_bf16(dst_ref[...])
    w12 = w12_ref[...]
    w3 = w3_ref[...]
    bias = b_ref[...]

    def stream(sl):
        x = jnp.concatenate([sev[:, sl], sod[:, sl], dev[:, sl], dod[:, sl]],
                            axis=1)
        return jnp.dot(x, w12, preferred_element_type=jnp.float32)

    ze = stream(slice(0, HWORDS))
    zo = stream(slice(HWORDS, 2 * HWORDS))
    ze = ze + jnp.dot(ef0_ref[...].astype(jnp.bfloat16), w3,
                      preferred_element_type=jnp.float32) + bias
    zo = zo + jnp.dot(ef1_ref[...].astype(jnp.bfloat16), w3,
                      preferred_element_type=jnp.float32) + bias
    return ze, zo


_EDGE_SPECS = [
    pl.BlockSpec((HTILE, 2 * HWORDS), lambda i: (i, 0)),
    pl.BlockSpec((HTILE, 2 * HWORDS), lambda i: (i, 0)),
    pl.BlockSpec((HTILE, EDGE_DIM), lambda i: (i, 0)),
    pl.BlockSpec((HTILE, EDGE_DIM), lambda i: (i, 0)),
    pl.BlockSpec((2 * HIDDEN, OUT_DIM), lambda i: (0, 0)),
    pl.BlockSpec((EDGE_DIM, OUT_DIM), lambda i: (0, 0)),
    pl.BlockSpec((1, OUT_DIM), lambda i: (0, 0)),
]


def _pass_a(src2, dst2, ef0, ef1, w12p, w3t, b_row):
    def body(src_ref, dst_ref, ef0_ref, ef1_ref, w12_ref, w3_ref, b_ref,
             s1_ref, s2_ref):
        ze, zo = _z_pair(src_ref, dst_ref, ef0_ref, ef1_ref,
                         w12_ref, w3_ref, b_ref)
        s1 = jnp.sum(ze, axis=0, keepdims=True) + \
            jnp.sum(zo, axis=0, keepdims=True)
        s2 = jnp.sum(ze * ze, axis=0, keepdims=True) + \
            jnp.sum(zo * zo, axis=0, keepdims=True)
        s1_ref[...] = s1[None]
        s2_ref[...] = s2[None]

    return pl.pallas_call(
        body,
        grid=(N_TILES,),
        in_specs=_EDGE_SPECS,
        out_specs=[
            pl.BlockSpec((1, 1, OUT_DIM), lambda i: (i, 0, 0)),
            pl.BlockSpec((1, 1, OUT_DIM), lambda i: (i, 0, 0)),
        ],
        out_shape=[
            jax.ShapeDtypeStruct((N_TILES, 1, OUT_DIM), jnp.float32),
            jax.ShapeDtypeStruct((N_TILES, 1, OUT_DIM), jnp.float32),
        ],
    )(src2, dst2, ef0, ef1, w12p, w3t, b_row)


def _finalize(s1p, s2p, gamma_row, beta_row):
    def body(s1_ref, s2_ref, g_ref, be_ref, sc_ref, sh_ref):
        inv_n = jnp.float32(1.0 / N_EDGES)
        mean = jnp.sum(s1_ref[...], axis=0) * inv_n
        ex2 = jnp.sum(s2_ref[...], axis=0) * inv_n
        var = ex2 - mean * mean
        scale = g_ref[...] * jax.lax.rsqrt(var + BN_EPS)
        sc_ref[...] = scale
        sh_ref[...] = be_ref[...] - mean * scale

    return pl.pallas_call(
        body,
        out_shape=[
            jax.ShapeDtypeStruct((1, OUT_DIM), jnp.float32),
            jax.ShapeDtypeStruct((1, OUT_DIM), jnp.float32),
        ],
    )(s1p, s2p, gamma_row, beta_row)


def _pass_b(src2, dst2, ef0, ef1, w12p, w3t, b_row, scale, shift):
    def body(src_ref, dst_ref, ef0_ref, ef1_ref, w12_ref, w3_ref, b_ref,
             sc_ref, sh_ref, m0_ref, m1_ref):
        ze, zo = _z_pair(src_ref, dst_ref, ef0_ref, ef1_ref,
                         w12_ref, w3_ref, b_ref)
        sc = sc_ref[...]
        sh = sh_ref[...]

        def msg(z):
            zn = z * sc + sh
            return jax.nn.sigmoid(zn[:, :HIDDEN]) * \
                jax.nn.softplus(zn[:, HIDDEN:])

        m0_ref[...] = msg(ze)
        m1_ref[...] = msg(zo)

    return pl.pallas_call(
        body,
        grid=(N_TILES,),
        in_specs=_EDGE_SPECS + [
            pl.BlockSpec((1, OUT_DIM), lambda i: (0, 0)),
            pl.BlockSpec((1, OUT_DIM), lambda i: (0, 0)),
        ],
        out_specs=[
            pl.BlockSpec((HTILE, HIDDEN), lambda i: (i, 0)),
            pl.BlockSpec((HTILE, HIDDEN), lambda i: (i, 0)),
        ],
        out_shape=[
            jax.ShapeDtypeStruct((N_HALF, HIDDEN), jnp.float32),
            jax.ShapeDtypeStruct((N_HALF, HIDDEN), jnp.float32),
        ],
    )(src2, dst2, ef0, ef1, w12p, w3t, b_row, scale, shift)


def _sc_scatter(m0, m1, di0, di1):
    @functools.partial(
        pl.kernel,
        out_type=jax.ShapeDtypeStruct((NUM_CORES, N_NODES_PAD, HIDDEN),
                                      jnp.float32),
        mesh=_sc_mesh(),
        scratch_types=[
            pltpu.VMEM_SHARED((N_NODES_PAD, HIDDEN), jnp.float32),
            pltpu.VMEM((CHUNK, HIDDEN), jnp.float32),
            pltpu.VMEM((1, CHUNK), jnp.int32),
            pltpu.VMEM((ZROWS, HIDDEN), jnp.float32),
        ],
    )
    def k(m0_hbm, m1_hbm, di0_hbm, di1_hbm, out_hbm, acc_sh, m_v, idx_v, z_v):
        cid = jax.lax.axis_index("core")
        sid = jax.lax.axis_index("subcore")

        zvec = jnp.zeros((16,), jnp.float32)

        @pl.loop(0, ZROWS)
        def _(r):
            @pl.loop(0, HIDDEN, step=16)
            def _(c0):
                z_v[r, pl.ds(c0, 16)] = zvec

        my_rows = sid * ROWS_PER_SUBCORE

        @pl.loop(0, ROWS_PER_SUBCORE, step=ZROWS)
        def _(r0):
            pltpu.sync_copy(z_v, acc_sh.at[pl.ds(my_rows + r0, ZROWS)])

        plsc.subcore_barrier()

        wid = sid * NUM_CORES + cid

        def do_chunk(m_hbm, di_hbm, c):
            pltpu.sync_copy(di_hbm.at[c], idx_v.at[0])
            pltpu.sync_copy(m_hbm.at[pl.ds(c * CHUNK, CHUNK)], m_v)
            pltpu.sync_copy(m_v, acc_sh.at[idx_v.at[0]], add=True)

        @pl.loop(0, CHUNKS_PER_WORKER)
        def _(j):
            do_chunk(m0_hbm, di0_hbm, wid * CHUNKS_PER_WORKER + j)

        @pl.loop(0, CHUNKS_PER_WORKER)
        def _(j):
            do_chunk(m1_hbm, di1_hbm, wid * CHUNKS_PER_WORKER + j)

        @pl.when(wid < REM_CHUNKS)
        def _():
            do_chunk(m0_hbm, di0_hbm, NUM_WORKERS * CHUNKS_PER_WORKER + wid)

        @pl.when(wid < REM_CHUNKS)
        def _():
            do_chunk(m1_hbm, di1_hbm, NUM_WORKERS * CHUNKS_PER_WORKER + wid)

        plsc.subcore_barrier()
        pltpu.sync_copy(
            acc_sh.at[pl.ds(my_rows, ROWS_PER_SUBCORE)],
            out_hbm.at[cid, pl.ds(my_rows, ROWS_PER_SUBCORE)])

    return k(m0, m1, di0, di1)


def _final(node_feats, partials):
    tile = 1000

    def body(nf_ref, p_ref, o_ref):
        o_ref[...] = jax.nn.softplus(nf_ref[...] + p_ref[0] + p_ref[1])

    return pl.pallas_call(
        body,
        grid=(N_NODES // tile,),
        in_specs=[
            pl.BlockSpec((tile, HIDDEN), lambda i: (i, 0)),
            pl.BlockSpec((NUM_CORES, tile, HIDDEN), lambda i: (0, i, 0)),
        ],
        out_specs=pl.BlockSpec((tile, HIDDEN), lambda i: (i, 0)),
        out_shape=jax.ShapeDtypeStruct((N_NODES, HIDDEN), jnp.float32),
    )(node_feats, partials)


def kernel(node_feats, edge_feats, edge_index, W, b, gamma, beta):
    edge_index = edge_index.astype(jnp.int32)
    src_idx = edge_index[0].reshape(1, N_EDGES)
    dst_idx = edge_index[1].reshape(1, N_EDGES)

    nf16 = node_feats.astype(jnp.bfloat16)
    u16 = jax.lax.bitcast_convert_type(nf16, jnp.uint16)
    packed = (u16[:, 0::2].astype(jnp.uint32)
              | (u16[:, 1::2].astype(jnp.uint32) << 16))
    packed = jax.lax.bitcast_convert_type(packed, jnp.int32)

    ef0 = edge_feats[0::2]
    ef1 = edge_feats[1::2]
    di0 = edge_index[1, 0::2].reshape(HALF_CHUNKS, CHUNK)
    di1 = edge_index[1, 1::2].reshape(HALF_CHUNKS, CHUNK)

    w12p = W[:, :2 * HIDDEN].T.astype(jnp.bfloat16)[_PERM]
    w3t = W[:, 2 * HIDDEN:].T.astype(jnp.bfloat16)
    b_row = b.reshape(1, OUT_DIM)
    gamma_row = gamma.reshape(1, OUT_DIM)
    beta_row = beta.reshape(1, OUT_DIM)

    src2, dst2 = _sc_gather(packed, src_idx, dst_idx)
    s1p, s2p = _pass_a(src2, dst2, ef0, ef1, w12p, w3t, b_row)
    scale, shift = _finalize(s1p, s2p, gamma_row, beta_row)
    m0, m1 = _pass_b(src2, dst2, ef0, ef1, w12p, w3t, b_row, scale, shift)
    partials = _sc_scatter(m0, m1, di0, di1)
    return _final(node_feats, partials)


# speedup vs baseline: 1.5336x; 1.5336x over previous
import functools

import jax
import jax.numpy as jnp
import numpy as np
from jax.experimental import pallas as pl
from jax.experimental.pallas import tpu as pltpu
from jax.experimental.pallas import tpu_sc as plsc

N_NODES = 10000
N_EDGES = 320000
N_HALF = N_EDGES // 2
HIDDEN = 128
HWORDS = HIDDEN // 2
EDGE_DIM = 16
OUT_DIM = 2 * HIDDEN
BN_EPS = 1e-5

NUM_CORES = 2
NUM_SUBCORES = 16
NUM_WORKERS = NUM_CORES * NUM_SUBCORES

GATHER_W = 128
EDGE_TILE = 1280
HTILE = EDGE_TILE // 2
N_TILES = N_EDGES // EDGE_TILE

CHUNK = 128
HALF_CHUNKS = N_HALF // CHUNK
CHUNKS_PER_WORKER = HALF_CHUNKS // NUM_WORKERS
REM_CHUNKS = HALF_CHUNKS - CHUNKS_PER_WORKER * NUM_WORKERS
N_NODES_PAD = 10240
ROWS_PER_SUBCORE = N_NODES_PAD // NUM_SUBCORES
ZROWS = 128

_PERM = np.concatenate([np.arange(0, 128, 2), np.arange(1, 128, 2),
                        np.arange(128, 256, 2), np.arange(129, 256, 2)])


def _sc_mesh():
    return plsc.VectorSubcoreMesh(core_axis_name="core",
                                  subcore_axis_name="subcore")


G_WIN = 128
G_WINDOWS = N_HALF // G_WIN
G_WPW = G_WINDOWS // NUM_WORKERS
G_REM = G_WINDOWS - G_WPW * NUM_WORKERS
G_IDX = (G_WPW + 1) * G_WIN


def _sc_gather(packed, si0, si1, di0, di1):
    out_t = jax.ShapeDtypeStruct((N_HALF, 2 * HWORDS), jnp.int32)

    @functools.partial(
        pl.kernel, out_type=(out_t, out_t), mesh=_sc_mesh(),
        compiler_params=pltpu.CompilerParams(use_tc_tiling_on_sc=False),
        scratch_types=[
            pltpu.VMEM((4, G_IDX), jnp.int32),
            pltpu.VMEM((2, 4, G_WIN, HWORDS), jnp.int32),
            pltpu.SemaphoreType.DMA,
            pltpu.SemaphoreType.DMA,
            pltpu.SemaphoreType.DMA,
            pltpu.SemaphoreType.DMA,
        ],
    )
    def k(nf_hbm, si0_hbm, si1_hbm, di0_hbm, di1_hbm, os_hbm, od_hbm,
          idx_v, g_v, sg0, sg1, sw0, sw1):
        cid = jax.lax.axis_index("core")
        sid = jax.lax.axis_index("subcore")
        wid = sid * NUM_CORES + cid
        n_win = G_WPW + (wid < G_REM).astype(jnp.int32)

        sgs = (sg0, sg1)
        sws = (sw0, sw1)
        idx_hbms = (si0_hbm, si1_hbm, di0_hbm, di1_hbm)
        outs = ((os_hbm, 0), (os_hbm, HWORDS), (od_hbm, 0), (od_hbm, HWORDS))

        for s in range(4):
            pltpu.sync_copy(idx_hbms[s].at[0, pl.ds(wid * G_WPW * G_WIN,
                                                    G_WPW * G_WIN)],
                            idx_v.at[s, pl.ds(0, G_WPW * G_WIN)])

        @pl.when(wid < G_REM)
        def _():
            for s in range(4):
                pltpu.sync_copy(
                    idx_hbms[s].at[0, pl.ds((G_WINDOWS - G_REM + wid) * G_WIN,
                                            G_WIN)],
                    idx_v.at[s, pl.ds(G_WPW * G_WIN, G_WIN)])

        def idx_off(w):
            return jnp.where(w < G_WPW, w * G_WIN, G_WPW * G_WIN)

        def row_base(w):
            return jnp.where(w < G_WPW, (wid * G_WPW + w) * G_WIN,
                             (G_WINDOWS - G_REM + wid) * G_WIN)

        def issue_gathers(w, b):
            off = idx_off(w)
            for s in range(4):
                pltpu.async_copy(
                    nf_hbm.at[idx_v.at[s, pl.ds(off, G_WIN)]],
                    g_v.at[b, s], sgs[b])

        def drain_gathers(b):
            for s in range(4):
                pltpu.make_async_copy(
                    nf_hbm.at[pl.ds(0, G_WIN)], g_v.at[b, s], sgs[b]).wait()

        def issue_writes(w, b):
            base = row_base(w)
            for s in range(4):
                o_hbm, c0 = outs[s]
                pltpu.async_copy(
                    g_v.at[b, s],
                    o_hbm.at[pl.ds(base, G_WIN), pl.ds(c0, HWORDS)], sws[b])

        def drain_writes(w, b):
            base = row_base(w)
            for s in range(4):
                o_hbm, c0 = outs[s]
                pltpu.make_async_copy(
                    g_v.at[b, s],
                    o_hbm.at[pl.ds(base, G_WIN), pl.ds(c0, HWORDS)],
                    sws[b]).wait()

        issue_gathers(jnp.int32(0), 0)
        issue_gathers(jnp.int32(1), 1)

        @pl.loop(0, G_WPW + 1, step=2)
        def _(j0):
            for b in (0, 1):
                w = j0 + b

                @pl.when(w < n_win)
                def _():
                    drain_gathers(b)
                    issue_writes(w, b)
                    drain_writes(w, b)

                    @pl.when(w + 2 < n_win)
                    def _():
                        issue_gathers(w + 2, b)

    return k(packed, si0, si1, di0, di1)


def _unpack_bf16(xi):
    ev = pltpu.bitcast(xi << 16, jnp.float32).astype(jnp.bfloat16)
    od = pltpu.bitcast(xi & jnp.int32(-65536), jnp.float32).astype(
        jnp.bfloat16)
    return ev, od


def _z_pair(src_ref, dst_ref, ef0_ref, ef1_ref, w12_ref, w3_ref, b_ref):
    sev, sod = _unpack_bf16(src_ref[...])
    dev, dod = _unpack_bf16(dst_ref[...])
    w12 = w12_ref[...]
    w3 = w3_ref[...]
    bias = b_ref[...]

    def stream(sl):
        x = jnp.concatenate([sev[:, sl], sod[:, sl], dev[:, sl], dod[:, sl]],
                            axis=1)
        return jnp.dot(x, w12, preferred_element_type=jnp.float32)

    ze = stream(slice(0, HWORDS))
    zo = stream(slice(HWORDS, 2 * HWORDS))
    ze = ze + jnp.dot(ef0_ref[...].astype(jnp.bfloat16), w3,
                      preferred_element_type=jnp.float32) + bias
    zo = zo + jnp.dot(ef1_ref[...].astype(jnp.bfloat16), w3,
                      preferred_element_type=jnp.float32) + bias
    return ze, zo


_EDGE_SPECS = [
    pl.BlockSpec((HTILE, 2 * HWORDS), lambda i: (i, 0)),
    pl.BlockSpec((HTILE, 2 * HWORDS), lambda i: (i, 0)),
    pl.BlockSpec((HTILE, EDGE_DIM), lambda i: (i, 0)),
    pl.BlockSpec((HTILE, EDGE_DIM), lambda i: (i, 0)),
    pl.BlockSpec((2 * HIDDEN, OUT_DIM), lambda i: (0, 0)),
    pl.BlockSpec((EDGE_DIM, OUT_DIM), lambda i: (0, 0)),
    pl.BlockSpec((1, OUT_DIM), lambda i: (0, 0)),
]


def _pass_a(src2, dst2, ef0, ef1, w12p, w3t, b_row):
    def body(src_ref, dst_ref, ef0_ref, ef1_ref, w12_ref, w3_ref, b_ref,
             s1_ref, s2_ref):
        ze, zo = _z_pair(src_ref, dst_ref, ef0_ref, ef1_ref,
                         w12_ref, w3_ref, b_ref)
        s1 = jnp.sum(ze, axis=0, keepdims=True) + \
            jnp.sum(zo, axis=0, keepdims=True)
        s2 = jnp.sum(ze * ze, axis=0, keepdims=True) + \
            jnp.sum(zo * zo, axis=0, keepdims=True)
        s1_ref[...] = s1[None]
        s2_ref[...] = s2[None]

    return pl.pallas_call(
        body,
        grid=(N_TILES,),
        in_specs=_EDGE_SPECS,
        out_specs=[
            pl.BlockSpec((1, 1, OUT_DIM), lambda i: (i, 0, 0)),
            pl.BlockSpec((1, 1, OUT_DIM), lambda i: (i, 0, 0)),
        ],
        out_shape=[
            jax.ShapeDtypeStruct((N_TILES, 1, OUT_DIM), jnp.float32),
            jax.ShapeDtypeStruct((N_TILES, 1, OUT_DIM), jnp.float32),
        ],
    )(src2, dst2, ef0, ef1, w12p, w3t, b_row)


def _finalize(s1p, s2p, gamma_row, beta_row):
    def body(s1_ref, s2_ref, g_ref, be_ref, sc_ref, sh_ref):
        inv_n = jnp.float32(1.0 / N_EDGES)
        mean = jnp.sum(s1_ref[...], axis=0) * inv_n
        ex2 = jnp.sum(s2_ref[...], axis=0) * inv_n
        var = ex2 - mean * mean
        scale = g_ref[...] * jax.lax.rsqrt(var + BN_EPS)
        sc_ref[...] = scale
        sh_ref[...] = be_ref[...] - mean * scale

    return pl.pallas_call(
        body,
        out_shape=[
            jax.ShapeDtypeStruct((1, OUT_DIM), jnp.float32),
            jax.ShapeDtypeStruct((1, OUT_DIM), jnp.float32),
        ],
    )(s1p, s2p, gamma_row, beta_row)


def _pass_b(src2, dst2, ef0, ef1, w12p, w3t, b_row, scale, shift):
    def body(src_ref, dst_ref, ef0_ref, ef1_ref, w12_ref, w3_ref, b_ref,
             sc_ref, sh_ref, m0_ref, m1_ref):
        ze, zo = _z_pair(src_ref, dst_ref, ef0_ref, ef1_ref,
                         w12_ref, w3_ref, b_ref)
        sc = sc_ref[...]
        sh = sh_ref[...]

        def msg(z):
            zn = z * sc + sh
            return jax.nn.sigmoid(zn[:, :HIDDEN]) * \
                jax.nn.softplus(zn[:, HIDDEN:])

        m0_ref[...] = msg(ze)
        m1_ref[...] = msg(zo)

    return pl.pallas_call(
        body,
        grid=(N_TILES,),
        in_specs=_EDGE_SPECS + [
            pl.BlockSpec((1, OUT_DIM), lambda i: (0, 0)),
            pl.BlockSpec((1, OUT_DIM), lambda i: (0, 0)),
        ],
        out_specs=[
            pl.BlockSpec((HTILE, HIDDEN), lambda i: (i, 0)),
            pl.BlockSpec((HTILE, HIDDEN), lambda i: (i, 0)),
        ],
        out_shape=[
            jax.ShapeDtypeStruct((N_HALF, HIDDEN), jnp.float32),
            jax.ShapeDtypeStruct((N_HALF, HIDDEN), jnp.float32),
        ],
    )(src2, dst2, ef0, ef1, w12p, w3t, b_row, scale, shift)


def _sc_scatter(m0, m1, di0, di1):
    @functools.partial(
        pl.kernel,
        out_type=jax.ShapeDtypeStruct((NUM_CORES, N_NODES_PAD, HIDDEN),
                                      jnp.float32),
        mesh=_sc_mesh(),
        scratch_types=[
            pltpu.VMEM_SHARED((N_NODES_PAD, HIDDEN), jnp.float32),
            pltpu.VMEM((CHUNK, HIDDEN), jnp.float32),
            pltpu.VMEM((1, CHUNK), jnp.int32),
            pltpu.VMEM((ZROWS, HIDDEN), jnp.float32),
        ],
    )
    def k(m0_hbm, m1_hbm, di0_hbm, di1_hbm, out_hbm, acc_sh, m_v, idx_v, z_v):
        cid = jax.lax.axis_index("core")
        sid = jax.lax.axis_index("subcore")

        zvec = jnp.zeros((16,), jnp.float32)

        @pl.loop(0, ZROWS)
        def _(r):
            @pl.loop(0, HIDDEN, step=16)
            def _(c0):
                z_v[r, pl.ds(c0, 16)] = zvec

        my_rows = sid * ROWS_PER_SUBCORE

        @pl.loop(0, ROWS_PER_SUBCORE, step=ZROWS)
        def _(r0):
            pltpu.sync_copy(z_v, acc_sh.at[pl.ds(my_rows + r0, ZROWS)])

        plsc.subcore_barrier()

        wid = sid * NUM_CORES + cid

        def do_chunk(m_hbm, di_hbm, c):
            pltpu.sync_copy(di_hbm.at[c], idx_v.at[0])
            pltpu.sync_copy(m_hbm.at[pl.ds(c * CHUNK, CHUNK)], m_v)
            pltpu.sync_copy(m_v, acc_sh.at[idx_v.at[0]], add=True)

        @pl.loop(0, CHUNKS_PER_WORKER)
        def _(j):
            do_chunk(m0_hbm, di0_hbm, wid * CHUNKS_PER_WORKER + j)

        @pl.loop(0, CHUNKS_PER_WORKER)
        def _(j):
            do_chunk(m1_hbm, di1_hbm, wid * CHUNKS_PER_WORKER + j)

        @pl.when(wid < REM_CHUNKS)
        def _():
            do_chunk(m0_hbm, di0_hbm, NUM_WORKERS * CHUNKS_PER_WORKER + wid)

        @pl.when(wid < REM_CHUNKS)
        def _():
            do_chunk(m1_hbm, di1_hbm, NUM_WORKERS * CHUNKS_PER_WORKER + wid)

        plsc.subcore_barrier()
        pltpu.sync_copy(
            acc_sh.at[pl.ds(my_rows, ROWS_PER_SUBCORE)],
            out_hbm.at[cid, pl.ds(my_rows, ROWS_PER_SUBCORE)])

    return k(m0, m1, di0, di1)


def _final(node_feats, partials):
    tile = 1000

    def body(nf_ref, p_ref, o_ref):
        o_ref[...] = jax.nn.softplus(nf_ref[...] + p_ref[0] + p_ref[1])

    return pl.pallas_call(
        body,
        grid=(N_NODES // tile,),
        in_specs=[
            pl.BlockSpec((tile, HIDDEN), lambda i: (i, 0)),
            pl.BlockSpec((NUM_CORES, tile, HIDDEN), lambda i: (0, i, 0)),
        ],
        out_specs=pl.BlockSpec((tile, HIDDEN), lambda i: (i, 0)),
        out_shape=jax.ShapeDtypeStruct((N_NODES, HIDDEN), jnp.float32),
    )(node_feats, partials)


def kernel(node_feats, edge_feats, edge_index, W, b, gamma, beta):
    edge_index = edge_index.astype(jnp.int32)
    src = edge_index[0]
    dst = edge_index[1]

    nf16 = node_feats.astype(jnp.bfloat16)
    u16 = jax.lax.bitcast_convert_type(nf16, jnp.uint16)
    packed = (u16[:, 0::2].astype(jnp.uint32)
              | (u16[:, 1::2].astype(jnp.uint32) << 16))
    packed = jax.lax.bitcast_convert_type(packed, jnp.int32)

    si0 = src[:N_HALF].reshape(1, N_HALF)
    si1 = src[N_HALF:].reshape(1, N_HALF)
    sd0 = dst[:N_HALF].reshape(1, N_HALF)
    sd1 = dst[N_HALF:].reshape(1, N_HALF)
    ef0 = edge_feats[:N_HALF]
    ef1 = edge_feats[N_HALF:]
    di0 = dst[:N_HALF].reshape(HALF_CHUNKS, CHUNK)
    di1 = dst[N_HALF:].reshape(HALF_CHUNKS, CHUNK)

    w12p = W[:, :2 * HIDDEN].T.astype(jnp.bfloat16)[_PERM]
    w3t = W[:, 2 * HIDDEN:].T.astype(jnp.bfloat16)
    b_row = b.reshape(1, OUT_DIM)
    gamma_row = gamma.reshape(1, OUT_DIM)
    beta_row = beta.reshape(1, OUT_DIM)

    src2, dst2 = _sc_gather(packed, si0, si1, sd0, sd1)
    s1p, s2p = _pass_a(src2, dst2, ef0, ef1, w12p, w3t, b_row)
    scale, shift = _finalize(s1p, s2p, gamma_row, beta_row)
    m0, m1 = _pass_b(src2, dst2, ef0, ef1, w12p, w3t, b_row, scale, shift)
    partials = _sc_scatter(m0, m1, di0, di1)
    return _final(node_feats, partials)


# speedup vs baseline: 1.9549x; 1.2747x over previous
import functools

import jax
import jax.numpy as jnp
from jax.experimental import pallas as pl
from jax.experimental.pallas import tpu as pltpu
from jax.experimental.pallas import tpu_sc as plsc

N_NODES = 10000
N_EDGES = 320000
HIDDEN = 128
EDGE_DIM = 16
OUT_DIM = 2 * HIDDEN
BN_EPS = 1e-5

NUM_CORES = 2
NUM_SUBCORES = 16
NUM_WORKERS = NUM_CORES * NUM_SUBCORES

N_CHUNKS_E = 4
CHUNK_E = N_EDGES // N_CHUNKS_E

GATHER_W = 128
EDGE_TILE = 1600
TILES_PER_CHUNK = CHUNK_E // EDGE_TILE

SCHUNK = 128
SC_CHUNKS = CHUNK_E // SCHUNK
SC_CPW = SC_CHUNKS // NUM_WORKERS
SC_REM = SC_CHUNKS - SC_CPW * NUM_WORKERS
N_NODES_PAD = 10240
ROWS_PER_SUBCORE = N_NODES_PAD // NUM_SUBCORES
ZROWS = 128


def _sc_mesh():
    return plsc.VectorSubcoreMesh(core_axis_name="core",
                                  subcore_axis_name="subcore")


def _sc_gather(node_feats, src_idx, dst_idx):
    out_t = jax.ShapeDtypeStruct((CHUNK_E, HIDDEN), node_feats.dtype)

    @functools.partial(pl.kernel, out_type=(out_t, out_t), mesh=_sc_mesh())
    def k(nf_hbm, si_hbm, di_hbm, os_hbm, od_hbm):
        def body(si_v, di_v, os_v, od_v):
            pltpu.sync_copy(nf_hbm.at[si_v.at[0]], os_v)
            pltpu.sync_copy(nf_hbm.at[di_v.at[0]], od_v)

        pltpu.emit_pipeline(
            body,
            grid=(CHUNK_E // GATHER_W,),
            in_specs=[
                pl.BlockSpec((1, GATHER_W), lambda i: (0, i)),
                pl.BlockSpec((1, GATHER_W), lambda i: (0, i)),
            ],
            out_specs=[
                pl.BlockSpec((GATHER_W, HIDDEN), lambda i: (i, 0)),
                pl.BlockSpec((GATHER_W, HIDDEN), lambda i: (i, 0)),
            ],
            core_axis_name=("core", "subcore"),
            dimension_semantics=(pltpu.PARALLEL,),
        )(si_hbm, di_hbm, os_hbm, od_hbm)

    return k(node_feats, src_idx, dst_idx)


def _pass_a(src_rows, dst_rows, edge_feats, w12t, w3t, b_row, chunk):
    base = chunk * TILES_PER_CHUNK

    def body(src_ref, dst_ref, ef_ref, w12_ref, w3_ref, b_ref,
             z_ref, s1_ref, s2_ref):
        x = jnp.concatenate([src_ref[...], dst_ref[...]], axis=1)
        z = jnp.dot(x.astype(jnp.bfloat16), w12_ref[...],
                    preferred_element_type=jnp.float32)
        z = z + jnp.dot(ef_ref[...].astype(jnp.bfloat16), w3_ref[...],
                        preferred_element_type=jnp.float32)
        z = z + b_ref[...]
        z_ref[...] = z.astype(jnp.bfloat16)
        s1_ref[...] = jnp.sum(z, axis=0, keepdims=True)[None]
        s2_ref[...] = jnp.sum(z * z, axis=0, keepdims=True)[None]

    return pl.pallas_call(
        body,
        grid=(TILES_PER_CHUNK,),
        in_specs=[
            pl.BlockSpec((EDGE_TILE, HIDDEN), lambda i: (i, 0)),
            pl.BlockSpec((EDGE_TILE, HIDDEN), lambda i: (i, 0)),
            pl.BlockSpec((EDGE_TILE, EDGE_DIM), lambda i: (base + i, 0)),
            pl.BlockSpec((2 * HIDDEN, OUT_DIM), lambda i: (0, 0)),
            pl.BlockSpec((EDGE_DIM, OUT_DIM), lambda i: (0, 0)),
            pl.BlockSpec((1, OUT_DIM), lambda i: (0, 0)),
        ],
        out_specs=[
            pl.BlockSpec((EDGE_TILE, OUT_DIM), lambda i: (i, 0)),
            pl.BlockSpec((1, 1, OUT_DIM), lambda i: (i, 0, 0)),
            pl.BlockSpec((1, 1, OUT_DIM), lambda i: (i, 0, 0)),
        ],
        out_shape=[
            jax.ShapeDtypeStruct((CHUNK_E, OUT_DIM), jnp.bfloat16),
            jax.ShapeDtypeStruct((TILES_PER_CHUNK, 1, OUT_DIM), jnp.float32),
            jax.ShapeDtypeStruct((TILES_PER_CHUNK, 1, OUT_DIM), jnp.float32),
        ],
    )(src_rows, dst_rows, edge_feats, w12t, w3t, b_row)


def _finalize(s1s, s2s, gamma_row, beta_row):
    def body(*refs):
        s_refs = refs[:N_CHUNKS_E]
        q_refs = refs[N_CHUNKS_E:2 * N_CHUNKS_E]
        g_ref, be_ref, sc_ref, sh_ref = refs[2 * N_CHUNKS_E:]
        inv_n = jnp.float32(1.0 / N_EDGES)
        s1 = sum(jnp.sum(r[...], axis=0) for r in s_refs)
        s2 = sum(jnp.sum(r[...], axis=0) for r in q_refs)
        mean = s1 * inv_n
        var = s2 * inv_n - mean * mean
        scale = g_ref[...] * jax.lax.rsqrt(var + BN_EPS)
        sc_ref[...] = scale
        sh_ref[...] = be_ref[...] - mean * scale

    return pl.pallas_call(
        body,
        out_shape=[
            jax.ShapeDtypeStruct((1, OUT_DIM), jnp.float32),
            jax.ShapeDtypeStruct((1, OUT_DIM), jnp.float32),
        ],
    )(*s1s, *s2s, gamma_row, beta_row)


def _pass_b(z_bf, scale, shift):
    def body(z_ref, sc_ref, sh_ref, m_ref):
        zn = z_ref[...].astype(jnp.float32) * sc_ref[...] + sh_ref[...]
        gate = zn[:, :HIDDEN]
        msg = zn[:, HIDDEN:]
        m_ref[...] = jax.nn.sigmoid(gate) * jax.nn.softplus(msg)

    return pl.pallas_call(
        body,
        grid=(TILES_PER_CHUNK,),
        in_specs=[
            pl.BlockSpec((EDGE_TILE, OUT_DIM), lambda i: (i, 0)),
            pl.BlockSpec((1, OUT_DIM), lambda i: (0, 0)),
            pl.BlockSpec((1, OUT_DIM), lambda i: (0, 0)),
        ],
        out_specs=pl.BlockSpec((EDGE_TILE, HIDDEN), lambda i: (i, 0)),
        out_shape=jax.ShapeDtypeStruct((CHUNK_E, HIDDEN), jnp.float32),
    )(z_bf, scale, shift)


def _sc_scatter(ma, mb, dia, dib):
    @functools.partial(
        pl.kernel,
        out_type=jax.ShapeDtypeStruct((NUM_CORES, N_NODES_PAD, HIDDEN),
                                      jnp.float32),
        mesh=_sc_mesh(),
        scratch_types=[
            pltpu.VMEM_SHARED((N_NODES_PAD, HIDDEN), jnp.float32),
            pltpu.VMEM((SCHUNK, HIDDEN), jnp.float32),
            pltpu.VMEM((1, SCHUNK), jnp.int32),
            pltpu.VMEM((ZROWS, HIDDEN), jnp.float32),
        ],
    )
    def k(ma_hbm, mb_hbm, dia_hbm, dib_hbm, out_hbm, acc_sh, m_v, idx_v, z_v):
        cid = jax.lax.axis_index("core")
        sid = jax.lax.axis_index("subcore")

        zvec = jnp.zeros((16,), jnp.float32)

        @pl.loop(0, ZROWS)
        def _(r):
            @pl.loop(0, HIDDEN, step=16)
            def _(c0):
                z_v[r, pl.ds(c0, 16)] = zvec

        my_rows = sid * ROWS_PER_SUBCORE

        @pl.loop(0, ROWS_PER_SUBCORE, step=ZROWS)
        def _(r0):
            pltpu.sync_copy(z_v, acc_sh.at[pl.ds(my_rows + r0, ZROWS)])

        plsc.subcore_barrier()

        wid = sid * NUM_CORES + cid

        def do_chunk(m_hbm, di_hbm, c):
            pltpu.sync_copy(di_hbm.at[c], idx_v.at[0])
            pltpu.sync_copy(m_hbm.at[pl.ds(c * SCHUNK, SCHUNK)], m_v)
            pltpu.sync_copy(m_v, acc_sh.at[idx_v.at[0]], add=True)

        for m_hbm, di_hbm in ((ma_hbm, dia_hbm), (mb_hbm, dib_hbm)):
            @pl.loop(0, SC_CPW)
            def _(j):
                do_chunk(m_hbm, di_hbm, wid * SC_CPW + j)

            @pl.when(wid < SC_REM)
            def _():
                do_chunk(m_hbm, di_hbm, NUM_WORKERS * SC_CPW + wid)

        plsc.subcore_barrier()
        pltpu.sync_copy(
            acc_sh.at[pl.ds(my_rows, ROWS_PER_SUBCORE)],
            out_hbm.at[cid, pl.ds(my_rows, ROWS_PER_SUBCORE)])

    return k(ma, mb, dia, dib)


def _final(node_feats, pa, pb):
    tile = 1000

    def body(nf_ref, pa_ref, pb_ref, o_ref):
        acc = nf_ref[...] + pa_ref[0] + pa_ref[1] + pb_ref[0] + pb_ref[1]
        o_ref[...] = jax.nn.softplus(acc)

    p_spec = pl.BlockSpec((NUM_CORES, tile, HIDDEN), lambda i: (0, i, 0))
    return pl.pallas_call(
        body,
        grid=(N_NODES // tile,),
        in_specs=[
            pl.BlockSpec((tile, HIDDEN), lambda i: (i, 0)),
            p_spec,
            p_spec,
        ],
        out_specs=pl.BlockSpec((tile, HIDDEN), lambda i: (i, 0)),
        out_shape=jax.ShapeDtypeStruct((N_NODES, HIDDEN), jnp.float32),
    )(node_feats, pa, pb)


def kernel(node_feats, edge_feats, edge_index, W, b, gamma, beta):
    edge_index = edge_index.astype(jnp.int32)
    src_idx = edge_index[0].reshape(1, N_EDGES)
    dst_idx = edge_index[1].reshape(1, N_EDGES)

    w12t = W[:, :2 * HIDDEN].T.astype(jnp.bfloat16)
    w3t = W[:, 2 * HIDDEN:].T.astype(jnp.bfloat16)
    b_row = b.reshape(1, OUT_DIM)
    gamma_row = gamma.reshape(1, OUT_DIM)
    beta_row = beta.reshape(1, OUT_DIM)

    zs, s1s, s2s = [], [], []
    for c in range(N_CHUNKS_E):
        lo = c * CHUNK_E
        src_c = jax.lax.dynamic_slice(src_idx, (0, lo), (1, CHUNK_E))
        dst_c = jax.lax.dynamic_slice(dst_idx, (0, lo), (1, CHUNK_E))
        sr, dr = _sc_gather(node_feats, src_c, dst_c)
        z_c, s1_c, s2_c = _pass_a(sr, dr, edge_feats, w12t, w3t, b_row, c)
        zs.append(z_c)
        s1s.append(s1_c)
        s2s.append(s2_c)

    scale, shift = _finalize(s1s, s2s, gamma_row, beta_row)

    msgs = [_pass_b(z_c, scale, shift) for z_c in zs]
    dis = [dst_idx[0, c * CHUNK_E:(c + 1) * CHUNK_E].reshape(SC_CHUNKS,
                                                            SCHUNK)
           for c in range(N_CHUNKS_E)]

    pa = _sc_scatter(msgs[0], msgs[1], dis[0], dis[1])
    pb = _sc_scatter(msgs[2], msgs[3], dis[2], dis[3])
    return _final(node_feats, pa, pb)


# speedup vs baseline: 2.1438x; 1.0966x over previous
import functools

import jax
import jax.numpy as jnp
from jax.experimental import pallas as pl
from jax.experimental.pallas import tpu as pltpu
from jax.experimental.pallas import tpu_sc as plsc

N_NODES = 10000
N_EDGES = 320000
HIDDEN = 128
EDGE_DIM = 16
OUT_DIM = 2 * HIDDEN
BN_EPS = 1e-5

NUM_CORES = 2
NUM_SUBCORES = 16
NUM_WORKERS = NUM_CORES * NUM_SUBCORES

N_CHUNKS_E = 4
CHUNK_E = N_EDGES // N_CHUNKS_E

GATHER_W = 128
EDGE_TILE = 1600
TILES_PER_CHUNK = CHUNK_E // EDGE_TILE

SCHUNK = 128
SC_CHUNKS = CHUNK_E // SCHUNK
SC_CPW = SC_CHUNKS // NUM_WORKERS
SC_REM = SC_CHUNKS - SC_CPW * NUM_WORKERS
N_NODES_PAD = 10240
ROWS_PER_SUBCORE = N_NODES_PAD // NUM_SUBCORES
ZROWS = 128


def _sc_mesh():
    return plsc.VectorSubcoreMesh(core_axis_name="core",
                                  subcore_axis_name="subcore")


def _sc_gather(node_feats, src_idx, dst_idx):
    out_t = jax.ShapeDtypeStruct((CHUNK_E, HIDDEN), node_feats.dtype)

    @functools.partial(pl.kernel, out_type=(out_t, out_t), mesh=_sc_mesh())
    def k(nf_hbm, si_hbm, di_hbm, os_hbm, od_hbm):
        def body(si_v, di_v, os_v, od_v):
            pltpu.sync_copy(nf_hbm.at[si_v.at[0]], os_v)
            pltpu.sync_copy(nf_hbm.at[di_v.at[0]], od_v)

        pltpu.emit_pipeline(
            body,
            grid=(CHUNK_E // GATHER_W,),
            in_specs=[
                pl.BlockSpec((1, GATHER_W), lambda i: (0, i)),
                pl.BlockSpec((1, GATHER_W), lambda i: (0, i)),
            ],
            out_specs=[
                pl.BlockSpec((GATHER_W, HIDDEN), lambda i: (i, 0)),
                pl.BlockSpec((GATHER_W, HIDDEN), lambda i: (i, 0)),
            ],
            core_axis_name=("core", "subcore"),
            dimension_semantics=(pltpu.PARALLEL,),
        )(si_hbm, di_hbm, os_hbm, od_hbm)

    return k(node_feats, src_idx, dst_idx)


def _pass_a(src_rows, dst_rows, edge_feats, w12t, w3t, b_row, chunk):
    base = chunk * TILES_PER_CHUNK

    def body(src_ref, dst_ref, ef_ref, w12_ref, w3_ref, b_ref,
             z_ref, s1_ref, s2_ref):
        x = jnp.concatenate([src_ref[...], dst_ref[...]], axis=1)
        z = jnp.dot(x.astype(jnp.bfloat16), w12_ref[...],
                    preferred_element_type=jnp.float32)
        z = z + jnp.dot(ef_ref[...], w3_ref[...],
                        preferred_element_type=jnp.float32)
        z = z + b_ref[...]
        z_ref[...] = z.astype(jnp.bfloat16)
        s1_ref[...] = jnp.sum(z, axis=0, keepdims=True)[None]
        s2_ref[...] = jnp.sum(z * z, axis=0, keepdims=True)[None]

    return pl.pallas_call(
        body,
        grid=(TILES_PER_CHUNK,),
        in_specs=[
            pl.BlockSpec((EDGE_TILE, HIDDEN), lambda i: (i, 0)),
            pl.BlockSpec((EDGE_TILE, HIDDEN), lambda i: (i, 0)),
            pl.BlockSpec((EDGE_TILE, EDGE_DIM), lambda i: (base + i, 0)),
            pl.BlockSpec((2 * HIDDEN, OUT_DIM), lambda i: (0, 0)),
            pl.BlockSpec((EDGE_DIM, OUT_DIM), lambda i: (0, 0)),
            pl.BlockSpec((1, OUT_DIM), lambda i: (0, 0)),
        ],
        out_specs=[
            pl.BlockSpec((EDGE_TILE, OUT_DIM), lambda i: (i, 0)),
            pl.BlockSpec((1, 1, OUT_DIM), lambda i: (i, 0, 0)),
            pl.BlockSpec((1, 1, OUT_DIM), lambda i: (i, 0, 0)),
        ],
        out_shape=[
            jax.ShapeDtypeStruct((CHUNK_E, OUT_DIM), jnp.bfloat16),
            jax.ShapeDtypeStruct((TILES_PER_CHUNK, 1, OUT_DIM), jnp.float32),
            jax.ShapeDtypeStruct((TILES_PER_CHUNK, 1, OUT_DIM), jnp.float32),
        ],
    )(src_rows, dst_rows, edge_feats, w12t, w3t, b_row)


def _finalize(s1s, s2s, gamma_row, beta_row):
    def body(*refs):
        s_refs = refs[:N_CHUNKS_E]
        q_refs = refs[N_CHUNKS_E:2 * N_CHUNKS_E]
        g_ref, be_ref, sc_ref, sh_ref = refs[2 * N_CHUNKS_E:]
        inv_n = jnp.float32(1.0 / N_EDGES)
        s1 = sum(jnp.sum(r[...], axis=0) for r in s_refs)
        s2 = sum(jnp.sum(r[...], axis=0) for r in q_refs)
        mean = s1 * inv_n
        var = s2 * inv_n - mean * mean
        scale = g_ref[...] * jax.lax.rsqrt(var + BN_EPS)
        sc_ref[...] = scale
        sh_ref[...] = be_ref[...] - mean * scale

    return pl.pallas_call(
        body,
        out_shape=[
            jax.ShapeDtypeStruct((1, OUT_DIM), jnp.float32),
            jax.ShapeDtypeStruct((1, OUT_DIM), jnp.float32),
        ],
    )(*s1s, *s2s, gamma_row, beta_row)


def _pass_b(z_bf, scale, shift):
    def body(z_ref, sc_ref, sh_ref, m_ref):
        zn = z_ref[...].astype(jnp.float32) * sc_ref[...] + sh_ref[...]
        gate = zn[:, :HIDDEN]
        msg = zn[:, HIDDEN:]
        m_ref[...] = jax.nn.sigmoid(gate) * jax.nn.softplus(msg)

    return pl.pallas_call(
        body,
        grid=(TILES_PER_CHUNK,),
        in_specs=[
            pl.BlockSpec((EDGE_TILE, OUT_DIM), lambda i: (i, 0)),
            pl.BlockSpec((1, OUT_DIM), lambda i: (0, 0)),
            pl.BlockSpec((1, OUT_DIM), lambda i: (0, 0)),
        ],
        out_specs=pl.BlockSpec((EDGE_TILE, HIDDEN), lambda i: (i, 0)),
        out_shape=jax.ShapeDtypeStruct((CHUNK_E, HIDDEN), jnp.float32),
    )(z_bf, scale, shift)


SC1_CPW = SC_CHUNKS // NUM_WORKERS
SC1_REM = SC_CHUNKS - SC1_CPW * NUM_WORKERS


def _sc_scatter1(m, di):
    @functools.partial(
        pl.kernel,
        out_type=jax.ShapeDtypeStruct((NUM_CORES, N_NODES_PAD, HIDDEN),
                                      jnp.float32),
        mesh=_sc_mesh(),
        scratch_types=[
            pltpu.VMEM_SHARED((N_NODES_PAD, HIDDEN), jnp.float32),
            pltpu.VMEM((SCHUNK, HIDDEN), jnp.float32),
            pltpu.VMEM((1, SCHUNK), jnp.int32),
            pltpu.VMEM((ZROWS, HIDDEN), jnp.float32),
        ],
    )
    def k(m_hbm, di_hbm, out_hbm, acc_sh, m_v, idx_v, z_v):
        cid = jax.lax.axis_index("core")
        sid = jax.lax.axis_index("subcore")

        zvec = jnp.zeros((16,), jnp.float32)

        @pl.loop(0, ZROWS)
        def _(r):
            @pl.loop(0, HIDDEN, step=16)
            def _(c0):
                z_v[r, pl.ds(c0, 16)] = zvec

        my_rows = sid * ROWS_PER_SUBCORE

        @pl.loop(0, ROWS_PER_SUBCORE, step=ZROWS)
        def _(r0):
            pltpu.sync_copy(z_v, acc_sh.at[pl.ds(my_rows + r0, ZROWS)])

        plsc.subcore_barrier()

        wid = sid * NUM_CORES + cid

        def do_chunk(c):
            pltpu.sync_copy(di_hbm.at[c], idx_v.at[0])
            pltpu.sync_copy(m_hbm.at[pl.ds(c * SCHUNK, SCHUNK)], m_v)
            pltpu.sync_copy(m_v, acc_sh.at[idx_v.at[0]], add=True)

        @pl.loop(0, SC1_CPW)
        def _(j):
            do_chunk(wid * SC1_CPW + j)

        @pl.when(wid < SC1_REM)
        def _():
            do_chunk(NUM_WORKERS * SC1_CPW + wid)

        plsc.subcore_barrier()
        pltpu.sync_copy(
            acc_sh.at[pl.ds(my_rows, ROWS_PER_SUBCORE)],
            out_hbm.at[cid, pl.ds(my_rows, ROWS_PER_SUBCORE)])

    return k(m, di)


def _sc_scatter(ma, mb, dia, dib):
    @functools.partial(
        pl.kernel,
        out_type=jax.ShapeDtypeStruct((NUM_CORES, N_NODES_PAD, HIDDEN),
                                      jnp.float32),
        mesh=_sc_mesh(),
        scratch_types=[
            pltpu.VMEM_SHARED((N_NODES_PAD, HIDDEN), jnp.float32),
            pltpu.VMEM((SCHUNK, HIDDEN), jnp.float32),
            pltpu.VMEM((1, SCHUNK), jnp.int32),
            pltpu.VMEM((ZROWS, HIDDEN), jnp.float32),
        ],
    )
    def k(ma_hbm, mb_hbm, dia_hbm, dib_hbm, out_hbm, acc_sh, m_v, idx_v, z_v):
        cid = jax.lax.axis_index("core")
        sid = jax.lax.axis_index("subcore")

        zvec = jnp.zeros((16,), jnp.float32)

        @pl.loop(0, ZROWS)
        def _(r):
            @pl.loop(0, HIDDEN, step=16)
            def _(c0):
                z_v[r, pl.ds(c0, 16)] = zvec

        my_rows = sid * ROWS_PER_SUBCORE

        @pl.loop(0, ROWS_PER_SUBCORE, step=ZROWS)
        def _(r0):
            pltpu.sync_copy(z_v, acc_sh.at[pl.ds(my_rows + r0, ZROWS)])

        plsc.subcore_barrier()

        wid = sid * NUM_CORES + cid

        def do_chunk(m_hbm, di_hbm, c):
            pltpu.sync_copy(di_hbm.at[c], idx_v.at[0])
            pltpu.sync_copy(m_hbm.at[pl.ds(c * SCHUNK, SCHUNK)], m_v)
            pltpu.sync_copy(m_v, acc_sh.at[idx_v.at[0]], add=True)

        for m_hbm, di_hbm in ((ma_hbm, dia_hbm), (mb_hbm, dib_hbm)):
            @pl.loop(0, SC_CPW)
            def _(j):
                do_chunk(m_hbm, di_hbm, wid * SC_CPW + j)

            @pl.when(wid < SC_REM)
            def _():
                do_chunk(m_hbm, di_hbm, NUM_WORKERS * SC_CPW + wid)

        plsc.subcore_barrier()
        pltpu.sync_copy(
            acc_sh.at[pl.ds(my_rows, ROWS_PER_SUBCORE)],
            out_hbm.at[cid, pl.ds(my_rows, ROWS_PER_SUBCORE)])

    return k(ma, mb, dia, dib)


def _final(node_feats, partials):
    tile = 1000

    def body(*refs):
        nf_ref = refs[0]
        p_refs = refs[1:-1]
        o_ref = refs[-1]
        acc = nf_ref[...]
        for p in p_refs:
            acc = acc + p[0] + p[1]
        o_ref[...] = jax.nn.softplus(acc)

    p_spec = pl.BlockSpec((NUM_CORES, tile, HIDDEN), lambda i: (0, i, 0))
    return pl.pallas_call(
        body,
        grid=(N_NODES // tile,),
        in_specs=[pl.BlockSpec((tile, HIDDEN), lambda i: (i, 0))]
        + [p_spec] * len(partials),
        out_specs=pl.BlockSpec((tile, HIDDEN), lambda i: (i, 0)),
        out_shape=jax.ShapeDtypeStruct((N_NODES, HIDDEN), jnp.float32),
    )(node_feats, *partials)


def kernel(node_feats, edge_feats, edge_index, W, b, gamma, beta):
    edge_index = edge_index.astype(jnp.int32)
    src_idx = edge_index[0].reshape(1, N_EDGES)
    dst_idx = edge_index[1].reshape(1, N_EDGES)

    w12t = W[:, :2 * HIDDEN].T.astype(jnp.bfloat16)
    w3t = W[:, 2 * HIDDEN:].T.astype(jnp.bfloat16)
    b_row = b.reshape(1, OUT_DIM)
    gamma_row = gamma.reshape(1, OUT_DIM)
    beta_row = beta.reshape(1, OUT_DIM)

    ef16 = edge_feats.astype(jnp.bfloat16)

    zs, s1s, s2s = [], [], []
    for c in range(N_CHUNKS_E):
        lo = c * CHUNK_E
        src_c = jax.lax.dynamic_slice(src_idx, (0, lo), (1, CHUNK_E))
        dst_c = jax.lax.dynamic_slice(dst_idx, (0, lo), (1, CHUNK_E))
        sr, dr = _sc_gather(node_feats, src_c, dst_c)
        z_c, s1_c, s2_c = _pass_a(sr, dr, ef16, w12t, w3t, b_row, c)
        zs.append(z_c)
        s1s.append(s1_c)
        s2s.append(s2_c)

    scale, shift = _finalize(s1s, s2s, gamma_row, beta_row)

    msgs = [_pass_b(z_c, scale, shift) for z_c in zs]
    dis = [dst_idx[0, c * CHUNK_E:(c + 1) * CHUNK_E].reshape(SC_CHUNKS,
                                                            SCHUNK)
           for c in range(N_CHUNKS_E)]

    partials = [_sc_scatter1(msgs[c], dis[c]) for c in range(N_CHUNKS_E)]
    return _final(node_feats, partials)


# speedup vs baseline: 2.1450x; 1.0005x over previous
import functools

import jax
import jax.numpy as jnp
from jax.experimental import pallas as pl
from jax.experimental.pallas import tpu as pltpu
from jax.experimental.pallas import tpu_sc as plsc

N_NODES = 10000
N_EDGES = 320000
HIDDEN = 128
EDGE_DIM = 16
OUT_DIM = 2 * HIDDEN
BN_EPS = 1e-5

NUM_CORES = 2
NUM_SUBCORES = 16
NUM_WORKERS = NUM_CORES * NUM_SUBCORES

N_CHUNKS_E = 4
CHUNK_E = N_EDGES // N_CHUNKS_E

GATHER_W = 128
EDGE_TILE = 1600
TILES_PER_CHUNK = CHUNK_E // EDGE_TILE

SCHUNK = 128
SC_CHUNKS = CHUNK_E // SCHUNK
N_NODES_PAD = 10240
ROWS_PER_SUBCORE = N_NODES_PAD // NUM_SUBCORES
ZROWS = 128


def _sc_mesh():
    return plsc.VectorSubcoreMesh(core_axis_name="core",
                                  subcore_axis_name="subcore")


def _sc_gather(node_feats, src_idx, dst_idx):
    out_t = jax.ShapeDtypeStruct((CHUNK_E, HIDDEN), node_feats.dtype)

    @functools.partial(pl.kernel, out_type=(out_t, out_t), mesh=_sc_mesh())
    def k(nf_hbm, si_hbm, di_hbm, os_hbm, od_hbm):
        def body(si_v, di_v, os_v, od_v):
            pltpu.sync_copy(nf_hbm.at[si_v.at[0]], os_v)
            pltpu.sync_copy(nf_hbm.at[di_v.at[0]], od_v)

        pltpu.emit_pipeline(
            body,
            grid=(CHUNK_E // GATHER_W,),
            in_specs=[
                pl.BlockSpec((1, GATHER_W), lambda i: (0, i)),
                pl.BlockSpec((1, GATHER_W), lambda i: (0, i)),
            ],
            out_specs=[
                pl.BlockSpec((GATHER_W, HIDDEN), lambda i: (i, 0)),
                pl.BlockSpec((GATHER_W, HIDDEN), lambda i: (i, 0)),
            ],
            core_axis_name=("core", "subcore"),
            dimension_semantics=(pltpu.PARALLEL,),
        )(si_hbm, di_hbm, os_hbm, od_hbm)

    return k(node_feats, src_idx, dst_idx)


def _pass_a(src_rows, dst_rows, edge_feats, w12t, w3t, b_row, chunk):
    base = chunk * TILES_PER_CHUNK

    def body(src_ref, dst_ref, ef_ref, w12_ref, w3_ref, b_ref,
             z_ref, s1_ref, s2_ref):
        x = jnp.concatenate([src_ref[...], dst_ref[...]], axis=1)
        z = jnp.dot(x.astype(jnp.bfloat16), w12_ref[...],
                    preferred_element_type=jnp.float32)
        z = z + jnp.dot(ef_ref[...], w3_ref[...],
                        preferred_element_type=jnp.float32)
        z = z + b_ref[...]
        z_ref[...] = z.astype(jnp.bfloat16)
        s1_ref[...] = jnp.sum(z, axis=0, keepdims=True)[None]
        s2_ref[...] = jnp.sum(z * z, axis=0, keepdims=True)[None]

    return pl.pallas_call(
        body,
        grid=(TILES_PER_CHUNK,),
        in_specs=[
            pl.BlockSpec((EDGE_TILE, HIDDEN), lambda i: (i, 0)),
            pl.BlockSpec((EDGE_TILE, HIDDEN), lambda i: (i, 0)),
            pl.BlockSpec((EDGE_TILE, EDGE_DIM), lambda i: (base + i, 0)),
            pl.BlockSpec((2 * HIDDEN, OUT_DIM), lambda i: (0, 0)),
            pl.BlockSpec((EDGE_DIM, OUT_DIM), lambda i: (0, 0)),
            pl.BlockSpec((1, OUT_DIM), lambda i: (0, 0)),
        ],
        out_specs=[
            pl.BlockSpec((EDGE_TILE, OUT_DIM), lambda i: (i, 0)),
            pl.BlockSpec((1, 1, OUT_DIM), lambda i: (i, 0, 0)),
            pl.BlockSpec((1, 1, OUT_DIM), lambda i: (i, 0, 0)),
        ],
        out_shape=[
            jax.ShapeDtypeStruct((CHUNK_E, OUT_DIM), jnp.bfloat16),
            jax.ShapeDtypeStruct((TILES_PER_CHUNK, 1, OUT_DIM), jnp.float32),
            jax.ShapeDtypeStruct((TILES_PER_CHUNK, 1, OUT_DIM), jnp.float32),
        ],
    )(src_rows, dst_rows, edge_feats, w12t, w3t, b_row)


def _finalize(s1s, s2s, gamma_row, beta_row):
    def body(*refs):
        s_refs = refs[:N_CHUNKS_E]
        q_refs = refs[N_CHUNKS_E:2 * N_CHUNKS_E]
        g_ref, be_ref, sc_ref, sh_ref = refs[2 * N_CHUNKS_E:]
        inv_n = jnp.float32(1.0 / N_EDGES)
        s1 = sum(jnp.sum(r[...], axis=0) for r in s_refs)
        s2 = sum(jnp.sum(r[...], axis=0) for r in q_refs)
        mean = s1 * inv_n
        var = s2 * inv_n - mean * mean
        scale = g_ref[...] * jax.lax.rsqrt(var + BN_EPS)
        sc_ref[...] = scale
        sh_ref[...] = be_ref[...] - mean * scale

    return pl.pallas_call(
        body,
        out_shape=[
            jax.ShapeDtypeStruct((1, OUT_DIM), jnp.float32),
            jax.ShapeDtypeStruct((1, OUT_DIM), jnp.float32),
        ],
    )(*s1s, *s2s, gamma_row, beta_row)


def _pass_b(z_bf, scale, shift):
    def body(z_ref, sc_ref, sh_ref, m_ref):
        zn = z_ref[...].astype(jnp.float32) * sc_ref[...] + sh_ref[...]
        gate = zn[:, :HIDDEN]
        msg = zn[:, HIDDEN:]
        m_ref[...] = jax.nn.sigmoid(gate) * jax.nn.softplus(msg)

    return pl.pallas_call(
        body,
        grid=(TILES_PER_CHUNK,),
        in_specs=[
            pl.BlockSpec((EDGE_TILE, OUT_DIM), lambda i: (i, 0)),
            pl.BlockSpec((1, OUT_DIM), lambda i: (0, 0)),
            pl.BlockSpec((1, OUT_DIM), lambda i: (0, 0)),
        ],
        out_specs=pl.BlockSpec((EDGE_TILE, HIDDEN), lambda i: (i, 0)),
        out_shape=jax.ShapeDtypeStruct((CHUNK_E, HIDDEN), jnp.float32),
    )(z_bf, scale, shift)


SC1_CPW = SC_CHUNKS // NUM_WORKERS
SC1_REM = SC_CHUNKS - SC1_CPW * NUM_WORKERS


def _sc_scatter1(m, di):
    @functools.partial(
        pl.kernel,
        out_type=jax.ShapeDtypeStruct((NUM_CORES, N_NODES_PAD, HIDDEN),
                                      jnp.float32),
        mesh=_sc_mesh(),
        scratch_types=[
            pltpu.VMEM_SHARED((N_NODES_PAD, HIDDEN), jnp.float32),
            pltpu.VMEM((SCHUNK, HIDDEN), jnp.float32),
            pltpu.VMEM((1, SCHUNK), jnp.int32),
            pltpu.VMEM((ZROWS, HIDDEN), jnp.float32),
        ],
    )
    def k(m_hbm, di_hbm, out_hbm, acc_sh, m_v, idx_v, z_v):
        cid = jax.lax.axis_index("core")
        sid = jax.lax.axis_index("subcore")

        zvec = jnp.zeros((16,), jnp.float32)

        @pl.loop(0, ZROWS)
        def _(r):
            @pl.loop(0, HIDDEN, step=16)
            def _(c0):
                z_v[r, pl.ds(c0, 16)] = zvec

        my_rows = sid * ROWS_PER_SUBCORE

        @pl.loop(0, ROWS_PER_SUBCORE, step=ZROWS)
        def _(r0):
            pltpu.sync_copy(z_v, acc_sh.at[pl.ds(my_rows + r0, ZROWS)])

        plsc.subcore_barrier()

        wid = sid * NUM_CORES + cid

        def do_chunk(c):
            pltpu.sync_copy(di_hbm.at[c], idx_v.at[0])
            pltpu.sync_copy(m_hbm.at[pl.ds(c * SCHUNK, SCHUNK)], m_v)
            pltpu.sync_copy(m_v, acc_sh.at[idx_v.at[0]], add=True)

        @pl.loop(0, SC1_CPW)
        def _(j):
            do_chunk(wid * SC1_CPW + j)

        @pl.when(wid < SC1_REM)
        def _():
            do_chunk(NUM_WORKERS * SC1_CPW + wid)

        plsc.subcore_barrier()
        pltpu.sync_copy(
            acc_sh.at[pl.ds(my_rows, ROWS_PER_SUBCORE)],
            out_hbm.at[cid, pl.ds(my_rows, ROWS_PER_SUBCORE)])

    return k(m, di)


def _final(node_feats, partials):
    tile = 1000

    def body(*refs):
        nf_ref = refs[0]
        p_refs = refs[1:-1]
        o_ref = refs[-1]
        acc = nf_ref[...]
        for p in p_refs:
            acc = acc + p[0] + p[1]
        o_ref[...] = jax.nn.softplus(acc)

    p_spec = pl.BlockSpec((NUM_CORES, tile, HIDDEN), lambda i: (0, i, 0))
    return pl.pallas_call(
        body,
        grid=(N_NODES // tile,),
        in_specs=[pl.BlockSpec((tile, HIDDEN), lambda i: (i, 0))]
        + [p_spec] * len(partials),
        out_specs=pl.BlockSpec((tile, HIDDEN), lambda i: (i, 0)),
        out_shape=jax.ShapeDtypeStruct((N_NODES, HIDDEN), jnp.float32),
    )(node_feats, *partials)


def kernel(node_feats, edge_feats, edge_index, W, b, gamma, beta):
    edge_index = edge_index.astype(jnp.int32)
    src_idx = edge_index[0].reshape(1, N_EDGES)
    dst_idx = edge_index[1].reshape(1, N_EDGES)

    w12t = W[:, :2 * HIDDEN].T.astype(jnp.bfloat16)
    w3t = W[:, 2 * HIDDEN:].T.astype(jnp.bfloat16)
    b_row = b.reshape(1, OUT_DIM)
    gamma_row = gamma.reshape(1, OUT_DIM)
    beta_row = beta.reshape(1, OUT_DIM)

    ef16 = edge_feats.astype(jnp.bfloat16)

    zs, s1s, s2s = [], [], []
    for c in range(N_CHUNKS_E):
        lo = c * CHUNK_E
        src_c = jax.lax.dynamic_slice(src_idx, (0, lo), (1, CHUNK_E))
        dst_c = jax.lax.dynamic_slice(dst_idx, (0, lo), (1, CHUNK_E))
        sr, dr = _sc_gather(node_feats, src_c, dst_c)
        z_c, s1_c, s2_c = _pass_a(sr, dr, ef16, w12t, w3t, b_row, c)
        zs.append(z_c)
        s1s.append(s1_c)
        s2s.append(s2_c)

    scale, shift = _finalize(s1s, s2s, gamma_row, beta_row)

    msgs = [_pass_b(z_c, scale, shift) for z_c in zs]
    dis = [dst_idx[0, c * CHUNK_E:(c + 1) * CHUNK_E].reshape(SC_CHUNKS,
                                                            SCHUNK)
           for c in range(N_CHUNKS_E)]

    partials = [_sc_scatter1(msgs[c], dis[c]) for c in range(N_CHUNKS_E)]
    return _final(node_feats, partials)
